# Initial kernel scaffold; baseline (speedup 1.0000x reference)
import jax
import jax.numpy as jnp
from jax.experimental import pallas as pl


def kernel(x, edge_index, edge_index2, Wl1, Wr1, att1, b1, Wl2, Wr2, att2, b2, dust_bin, sourceSize, targetSize):
    raise NotImplementedError("write your pallas kernel here")


# speedup vs baseline: 1.0005x; 1.0005x over previous
import functools

import jax
import jax.numpy as jnp
from jax.experimental import pallas as pl
from jax.experimental.pallas import tpu as pltpu

_N_NODES = 10000
_HID = 8
_IN_HEAD = 8
_NUM_CLASSES = 128
_SRC = 1024
_TGT = 1024

_NEG = -1e30
_MP = 1032
_NP = 1152


def _gat_dense(x, edge_index, Wl, Wr, att, bias, heads, out_ch, concat):
    N = x.shape[0]
    loops = jnp.arange(N, dtype=edge_index.dtype)
    src = jnp.concatenate([edge_index[0], loops])
    dst = jnp.concatenate([edge_index[1], loops])
    xl = (x @ Wl).reshape(N, heads, out_ch)
    xr = (x @ Wr).reshape(N, heads, out_ch)
    m = xl[src] + xr[dst]
    m_act = jax.nn.leaky_relu(m, 0.2)
    e = (m_act * att[None, :, :]).sum(-1)
    e_max = jax.ops.segment_max(e, dst, num_segments=N)
    e_max = jax.lax.stop_gradient(jnp.where(jnp.isfinite(e_max), e_max, 0.0))
    w = jnp.exp(e - e_max[dst])
    denom = jax.ops.segment_sum(w, dst, num_segments=N)
    alpha = w / (denom[dst] + 1e-16)
    out = jax.ops.segment_sum(xl[src] * alpha[:, :, None], dst, num_segments=N)
    if concat:
        out = out.reshape(N, heads * out_ch)
    else:
        out = out.mean(axis=1)
    return out + bias


def _ot_body(src_ref, tgt_ref, alpha_ref, out_ref, c_ref):
    s = src_ref[...]
    t = tgt_ref[...]
    alpha = alpha_ref[0]

    st = jax.lax.dot_general(s, t, (((1,), (1,)), ((), ())),
                             preferred_element_type=jnp.float32)
    d2 = ((s * s).sum(1, keepdims=True) + (t * t).sum(1, keepdims=True).T
          - 2.0 * st)

    c_ref[...] = jnp.full((_MP, _NP), _NEG, jnp.float32)
    c_ref[0:_SRC, 0:_TGT] = d2
    c_ref[0:_SRC, _TGT:_TGT + 1] = jnp.full((_SRC, 1), 1.0, jnp.float32) * alpha
    c_ref[_SRC:_SRC + 1, 0:_TGT] = jnp.full((1, _TGT), 1.0, jnp.float32) * alpha
    c_ref[_SRC:_SRC + 1, _TGT:_TGT + 1] = jnp.full((1, 1), 1.0, jnp.float32) * alpha
    C = c_ref[...]

    m, n = _SRC, _TGT
    norm = -jnp.log(jnp.float32(m + n))
    row_i = jax.lax.broadcasted_iota(jnp.int32, (_MP, 1), 0)
    col_j = jax.lax.broadcasted_iota(jnp.int32, (1, _NP), 1)
    row_mask = row_i <= m
    col_mask = col_j <= n
    log_mu = jnp.where(row_i == m, jnp.log(jnp.float32(n)) + norm, norm)
    log_mu = jnp.where(row_mask, log_mu, 0.0)
    log_nu = jnp.where(col_j == n, jnp.log(jnp.float32(m)) + norm, norm)
    log_nu = jnp.where(col_mask, log_nu, 0.0)

    def body(_, uv):
        u, v = uv
        x = C + v
        mx = jnp.max(x, axis=1, keepdims=True)
        u = log_mu - (mx + jnp.log(jnp.sum(jnp.exp(x - mx), axis=1, keepdims=True)))
        u = jnp.where(row_mask, u, _NEG)
        y = C + u
        my = jnp.max(y, axis=0, keepdims=True)
        v = log_nu - (my + jnp.log(jnp.sum(jnp.exp(y - my), axis=0, keepdims=True)))
        v = jnp.where(col_mask, v, _NEG)
        return (u, v)

    u0 = jnp.zeros((_MP, 1), jnp.float32)
    v0 = jnp.zeros((1, _NP), jnp.float32)
    u, v = jax.lax.fori_loop(0, 100, body, (u0, v0))
    Z = C + u + v - norm
    out_ref[...] = Z[0:_SRC + 1, 0:_TGT + 1]


@jax.jit
def _ot_pallas(source, target, dust_bin):
    return pl.pallas_call(
        _ot_body,
        out_shape=jax.ShapeDtypeStruct((_SRC + 1, _TGT + 1), jnp.float32),
        in_specs=[
            pl.BlockSpec(memory_space=pltpu.VMEM),
            pl.BlockSpec(memory_space=pltpu.VMEM),
            pl.BlockSpec(memory_space=pltpu.SMEM),
        ],
        out_specs=pl.BlockSpec(memory_space=pltpu.VMEM),
        scratch_shapes=[pltpu.VMEM((_MP, _NP), jnp.float32)],
    )(source, target, jnp.reshape(dust_bin, (1,)).astype(jnp.float32))


def kernel(x, edge_index, edge_index2, Wl1, Wr1, att1, b1, Wl2, Wr2, att2, b2,
           dust_bin, sourceSize, targetSize):
    h = x
    for _ in range(4):
        h = _gat_dense(h, edge_index, Wl1, Wr1, att1, b1, _IN_HEAD, _HID, True)
        h = jax.nn.elu(h)
        h = _gat_dense(h, edge_index2, Wl2, Wr2, att2, b2, 1, _NUM_CLASSES, False)
        h = jax.nn.log_softmax(h, axis=1)
    problem = jax.lax.stop_gradient(h)
    source = jax.lax.dynamic_slice_in_dim(problem, sourceSize - _SRC, _SRC, axis=0)
    target = jax.lax.dynamic_slice_in_dim(problem, sourceSize + (targetSize - _TGT),
                                          _TGT, axis=0)
    Z = _ot_pallas(source, target, dust_bin)
    return Z[None, :, :]
